# baseline (speedup 1.0000x reference)
import functools

import jax
import jax.numpy as jnp
from jax import lax
from jax.experimental import pallas as pl
from jax.experimental.pallas import tpu as pltpu
from jax.experimental.pallas import tpu_sc as plsc

_E = 8
_R = 8
_NEG = -3.0e38
_D = 2048
_NB = 4
_CH = _D // 16


def _sc_router_kernel(h_hbm, gw_hbm, gb_hbm, a2_hbm, b2_hbm,
                      delta_hbm, wt_hbm, st_hbm,
                      h_v, g_v, stage_v, l8_v, idx_v, arows_v, brows_v,
                      pd_v, wtbuf_v, wtsum_v, red_v, redi_v, slog_sh,
                      spart_sh, wpart_sh, sem):
    wid = lax.axis_index("s") * 2 + lax.axis_index("c")
    b = wid // 8
    k = lax.rem(wid, 8)
    iota16 = lax.iota(jnp.int32, 16)

    def allred(v, op):
        ref = red_v if v.dtype == jnp.float32 else redi_v
        for sh in (8, 4, 2, 1):
            ref[...] = v
            shifted = plsc.load_gather(ref, [lax.rem(iota16 + sh, 16)])
            v = op(v, shifted)
        return v

    pltpu.sync_copy(h_hbm.at[b], h_v)
    pltpu.sync_copy(gw_hbm.at[k], g_v)

    def dot_step(i, acc):
        return acc + h_v[pl.ds(i * 16, 16)] * g_v[pl.ds(i * 16, 16)]

    acc = lax.fori_loop(0, _CH, dot_step, jnp.zeros((16,), jnp.float32))
    stage_v[...] = allred(acc, jnp.add)
    pltpu.sync_copy(stage_v, slog_sh.at[wid])
    plsc.subcore_barrier()

    pltpu.sync_copy(slog_sh.at[pl.ds(b * 8, 8)], l8_v)
    rows = jnp.minimum(iota16, 7)
    lvec = plsc.load_gather(l8_v, [rows, iota16])
    pltpu.sync_copy(gb_hbm, stage_v)
    lvec = jnp.where(iota16 < 8, lvec + stage_v[...], _NEG)
    m1v = allred(lvec, jnp.maximum)
    i1v = allred(jnp.where(lvec == m1v, iota16, 99), jnp.minimum)
    masked = jnp.where(iota16 == i1v, _NEG, lvec)
    m2v = allred(masked, jnp.maximum)
    i2v = allred(jnp.where(masked == m2v, iota16, 99), jnp.minimum)
    ev = jnp.exp(m2v - m1v)
    w1v = 1.0 / (1.0 + ev)
    w2v = 1.0 - w1v

    idx_v[...] = jnp.where(iota16 < 8, i1v * 8 + iota16,
                           i2v * 8 + iota16 - 8)
    pltpu.async_copy(a2_hbm.at[idx_v], arows_v, sem).wait()
    pltpu.async_copy(b2_hbm.at[idx_v], brows_v, sem).wait()

    r0 = 2 * k
    r1 = 2 * k + 1

    def t_step(i, accs):
        a0, a1 = accs
        hv = h_v[pl.ds(i * 16, 16)]
        return (a0 + arows_v[r0, pl.ds(i * 16, 16)] * hv,
                a1 + arows_v[r1, pl.ds(i * 16, 16)] * hv)

    z16 = jnp.zeros((16,), jnp.float32)
    acc0, acc1 = lax.fori_loop(0, _CH, t_step, (z16, z16))
    t0v = allred(acc0, jnp.add)
    t1v = allred(acc1, jnp.add)
    r0v = jnp.full((16,), r0, jnp.int32)
    r1v = jnp.full((16,), r1, jnp.int32)
    wt0v = t0v * (1.0 / _R) * jnp.where(r0v < 8, w1v, w2v)
    wt1v = t1v * (1.0 / _R) * jnp.where(r1v < 8, w1v, w2v)

    def pd_step(i, _):
        pd_v[pl.ds(i * 16, 16)] = (wt0v * brows_v[r0, pl.ds(i * 16, 16)]
                                   + wt1v * brows_v[r1, pl.ds(i * 16, 16)])
        return 0

    lax.fori_loop(0, _CH, pd_step, 0)
    pltpu.sync_copy(pd_v, spart_sh.at[wid])

    gpos0v = plsc.load_gather(idx_v, [lax.rem(r0v, 16)])
    gpos1v = plsc.load_gather(idx_v, [lax.rem(r1v, 16)])
    for c in range(4):
        wtbuf_v[pl.ds(c * 16, 16)] = (
            jnp.where(c * 16 + iota16 == gpos0v, wt0v, 0.0)
            + jnp.where(c * 16 + iota16 == gpos1v, wt1v, 0.0))
    pltpu.sync_copy(wtbuf_v, wpart_sh.at[wid])
    plsc.subcore_barrier()

    @pl.when(k == 0)
    def _():
        for j in range(1, 8):
            pltpu.sync_copy(spart_sh.at[b * 8 + j], g_v)

            def add_step(i, _):
                pd_v[pl.ds(i * 16, 16)] += g_v[pl.ds(i * 16, 16)]
                return 0

            lax.fori_loop(0, _CH, add_step, 0)
            pltpu.sync_copy(wpart_sh.at[b * 8 + j], wtsum_v)
            for c in range(4):
                wtbuf_v[pl.ds(c * 16, 16)] += wtsum_v[pl.ds(c * 16, 16)]
        pltpu.sync_copy(pd_v, delta_hbm.at[b])
        pltpu.sync_copy(wtbuf_v, wt_hbm.at[b])

        def s_step(i, accs):
            sd, sd2 = accs
            v = pd_v[pl.ds(i * 16, 16)]
            return sd + v, sd2 + v * v

        sd, sd2 = lax.fori_loop(0, _CH, s_step, (z16, z16))
        sdtv = allred(sd, jnp.add)
        sd2tv = allred(sd2, jnp.add)
        stage_v[...] = (jnp.where(iota16 == 0, sdtv, 0.0)
                        + jnp.where(iota16 == 1, sd2tv, 0.0))
        pltpu.sync_copy(stage_v, st_hbm.at[b])


def _sc_router(h, gate_W, gb16, A2, B2):
    mesh = plsc.VectorSubcoreMesh(core_axis_name="c", subcore_axis_name="s")
    f = functools.partial(
        pl.kernel, mesh=mesh,
        compiler_params=pltpu.CompilerParams(needs_layout_passes=False),
        out_type=[
            jax.ShapeDtypeStruct((_NB, _D), jnp.float32),
            jax.ShapeDtypeStruct((_NB, 64), jnp.float32),
            jax.ShapeDtypeStruct((_NB, 16), jnp.float32),
        ],
        scratch_types=[
            pltpu.VMEM((_D,), jnp.float32),
            pltpu.VMEM((_D,), jnp.float32),
            pltpu.VMEM((16,), jnp.float32),
            pltpu.VMEM((8, 16), jnp.float32),
            pltpu.VMEM((16,), jnp.int32),
            pltpu.VMEM((16, _D), jnp.float32),
            pltpu.VMEM((16, _D), jnp.float32),
            pltpu.VMEM((_D,), jnp.float32),
            pltpu.VMEM((64,), jnp.float32),
            pltpu.VMEM((64,), jnp.float32),
            pltpu.VMEM((16,), jnp.float32),
            pltpu.VMEM((16,), jnp.int32),
            pltpu.VMEM_SHARED((32, 16), jnp.float32),
            pltpu.VMEM_SHARED((32, _D), jnp.float32),
            pltpu.VMEM_SHARED((32, 64), jnp.float32),
            pltpu.SemaphoreType.DMA,
        ],
    )(_sc_router_kernel)
    return f(h, gate_W, gb16, A2, B2)


def _pass1_kernel(x_ref, w65_ref, h_ref, xb_ref, s2_ref, *, inv_s):
    s = pl.program_id(1)
    ns = pl.num_programs(1)
    xm = x_ref[0]

    @pl.when(s == 0)
    def _():
        h_ref[...] = jnp.zeros_like(h_ref)

    h_ref[...] += jnp.sum(xm, axis=0)[None, None, :]

    @pl.when(s == ns - 1)
    def _():
        h_ref[...] = h_ref[...] * inv_s

    xb_ref[0] = lax.dot_general(xm, w65_ref[...], (((1,), (0,)), ((), ())),
                                preferred_element_type=jnp.float32)
    s2_ref[0] = jnp.sum(xm * xm, axis=1, keepdims=True)


def _pass2_kernel(x_ref, xb_ref, s2_ref, delta_ref, wt_ref, st_ref,
                  gamma_ref, beta_ref, o_ref):
    D = x_ref.shape[2]
    xm = x_ref[0]
    xb = xb_ref[0]
    s1 = xb[:, 64:65]
    cross = jnp.sum(xb[:, 0:64] * wt_ref[0], axis=1, keepdims=True)
    sum_d = st_ref[0, 0, 0]
    sum_d2 = st_ref[0, 0, 1]
    mu = (s1 + sum_d) * (1.0 / D)
    var = ((s2_ref[0] + 2.0 * cross + sum_d2) * (1.0 / D) - mu * mu)
    rs = lax.rsqrt(var + 1e-5)
    o_ref[...] = (((xm + delta_ref[0] - mu) * rs) * gamma_ref[...]
                  + beta_ref[...])[None]


def kernel(x, gate_W, gate_b, A_all, B_all, gamma, beta):
    B, S, D = x.shape
    s_blk = 512
    ns = S // s_blk

    A2 = A_all.reshape(_E * _R, D)
    B2 = jnp.transpose(B_all, (0, 2, 1)).reshape(_E * _R, D)
    w65 = jnp.concatenate([B2.T, jnp.ones((D, 1), jnp.float32)], axis=1)
    gb16 = jnp.pad(gate_b, (0, 8))
    gm = gamma.reshape(1, D)
    bt = beta.reshape(1, D)

    h, xb, s2 = pl.pallas_call(
        functools.partial(_pass1_kernel, inv_s=1.0 / S),
        grid=(B, ns),
        in_specs=[
            pl.BlockSpec((1, s_blk, D), lambda b, s: (b, s, 0)),
            pl.BlockSpec((D, _E * _R + 1), lambda b, s: (0, 0)),
        ],
        out_specs=[
            pl.BlockSpec((1, 1, D), lambda b, s: (b, 0, 0)),
            pl.BlockSpec((1, s_blk, _E * _R + 1), lambda b, s: (b, s, 0)),
            pl.BlockSpec((1, s_blk, 1), lambda b, s: (b, s, 0)),
        ],
        out_shape=[
            jax.ShapeDtypeStruct((B, 1, D), jnp.float32),
            jax.ShapeDtypeStruct((B, S, _E * _R + 1), jnp.float32),
            jax.ShapeDtypeStruct((B, S, 1), jnp.float32),
        ],
    )(x, w65)

    delta, wt, st = _sc_router(h.reshape(B, D), gate_W, gb16, A2, B2)

    out = pl.pallas_call(
        _pass2_kernel,
        grid=(B, ns),
        in_specs=[
            pl.BlockSpec((1, s_blk, D), lambda b, s: (b, s, 0)),
            pl.BlockSpec((1, s_blk, _E * _R + 1), lambda b, s: (b, s, 0)),
            pl.BlockSpec((1, s_blk, 1), lambda b, s: (b, s, 0)),
            pl.BlockSpec((1, 1, D), lambda b, s: (b, 0, 0)),
            pl.BlockSpec((1, 1, 64), lambda b, s: (b, 0, 0)),
            pl.BlockSpec((1, 1, 16), lambda b, s: (b, 0, 0)),
            pl.BlockSpec((1, D), lambda b, s: (0, 0)),
            pl.BlockSpec((1, D), lambda b, s: (0, 0)),
        ],
        out_specs=pl.BlockSpec((1, s_blk, D), lambda b, s: (b, s, 0)),
        out_shape=jax.ShapeDtypeStruct((B, S, D), jnp.float32),
    )(x, xb, s2, delta.reshape(B, 1, D), wt.reshape(B, 1, 64),
      st.reshape(B, 1, 16), gm, bt)
    return out


# speedup vs baseline: 2.0058x; 2.0058x over previous
import functools

import jax
import jax.numpy as jnp
from jax import lax
from jax.experimental import pallas as pl
from jax.experimental.pallas import tpu as pltpu

_E = 8
_R = 8
_NEG = -3.0e38


def _fused_kernel(x_ref, w65_ref, gw_ref, gb_ref, a2_ref, b2_ref,
                  gamma_ref, beta_ref, o_ref,
                  xc_ref, h_ref, xb_ref, s2_ref, delta_ref, wt_ref, st_ref,
                  sem, *, s_blk, ns, n_b, inv_s):
    b = pl.program_id(0)
    s = pl.program_id(1)
    D = x_ref.shape[2]
    slot = lax.rem(b, 2)
    pslot = 1 - slot

    def chunk_copy(bb, sl, j):
        return pltpu.make_async_copy(
            x_ref.at[bb, pl.ds(j * s_blk, s_blk), :],
            xc_ref.at[sl, pl.ds(j * s_blk, s_blk), :],
            sem.at[sl, j])

    @pl.when(jnp.logical_and(b == 0, s == 0))
    def _():
        for j in range(ns):
            chunk_copy(0, 0, j).start()

    @pl.when(jnp.logical_and(s == 0, jnp.logical_and(b >= 1, b < n_b)))
    def _():
        chunk_copy(b, slot, ns - 1).start()

    @pl.when(jnp.logical_and(s >= 1, b + 1 < n_b))
    def _():
        chunk_copy(b + 1, pslot, s - 1).start()

    @pl.when(b < n_b)
    def _stats():
        chunk_copy(b, slot, s).wait()
        xm = xc_ref[slot, pl.ds(s * s_blk, s_blk), :]

        @pl.when(s == 0)
        def _():
            h_ref[slot] = jnp.zeros((1, D), jnp.float32)

        h_ref[slot] += jnp.sum(xm, axis=0)[None, :]

        xb_ref[slot, pl.ds(s * s_blk, s_blk), :] = lax.dot_general(
            xm, w65_ref[...], (((1,), (0,)), ((), ())),
            preferred_element_type=jnp.float32)
        s2_ref[slot, pl.ds(s * s_blk, s_blk), :] = jnp.sum(
            xm * xm, axis=1, keepdims=True)

    @pl.when(b >= 1)
    def _normalize():
        @pl.when(s == 0)
        def _():
            h = h_ref[pslot] * inv_s
            logits = (jnp.sum(gw_ref[...] * h, axis=1, keepdims=True)
                      + gb_ref[...])
            iota8 = lax.broadcasted_iota(jnp.int32, (_E, 1), 0)
            m1 = jnp.max(logits)
            i1 = jnp.min(jnp.where(logits == m1, iota8, _E))
            masked = jnp.where(iota8 == i1, _NEG, logits)
            m2 = jnp.max(masked)
            i2 = jnp.min(jnp.where(masked == m2, iota8, _E))
            eb = jnp.exp(m2 - m1)
            denom = 1.0 + eb
            w1 = 1.0 / denom
            w2 = eb / denom
            t = jnp.sum(a2_ref[...] * h, axis=1, keepdims=True)
            e_ids = lax.broadcasted_iota(jnp.int32, (_E * _R, 1), 0) // _R
            wfull = (jnp.where(e_ids == i1, w1, 0.0)
                     + jnp.where(e_ids == i2, w2, 0.0))
            wt = wfull * t * (1.0 / _R)
            delta = jnp.sum(wt * b2_ref[...], axis=0, keepdims=True)
            delta_ref[...] = delta
            wt_ref[...] = wt.reshape(1, _E * _R)
            st_ref[0, 0] = jnp.sum(delta)
            st_ref[0, 1] = jnp.sum(delta * delta)

        xm = xc_ref[pslot, pl.ds(s * s_blk, s_blk), :]
        xb = xb_ref[pslot, pl.ds(s * s_blk, s_blk), :]
        s1 = xb[:, 64:65]
        cross = jnp.sum(xb[:, 0:64] * wt_ref[...], axis=1, keepdims=True)
        mu = (s1 + st_ref[0, 0]) * (1.0 / D)
        var = ((s2_ref[pslot, pl.ds(s * s_blk, s_blk), :] + 2.0 * cross
                + st_ref[0, 1]) * (1.0 / D) - mu * mu)
        rs = lax.rsqrt(var + 1e-5)
        o_ref[...] = (((xm + delta_ref[...] - mu) * rs) * gamma_ref[...]
                      + beta_ref[...])[None]


def kernel(x, gate_W, gate_b, A_all, B_all, gamma, beta):
    B, S, D = x.shape
    s_blk = 256
    ns = S // s_blk

    A2 = A_all.reshape(_E * _R, D)
    B2 = jnp.transpose(B_all, (0, 2, 1)).reshape(_E * _R, D)
    w65 = jnp.concatenate([B2.T, jnp.ones((D, 1), jnp.float32)], axis=1)
    gb = gate_b.reshape(_E, 1)
    gm = gamma.reshape(1, D)
    bt = beta.reshape(1, D)

    out = pl.pallas_call(
        functools.partial(_fused_kernel, s_blk=s_blk, ns=ns, n_b=B,
                          inv_s=1.0 / S),
        grid=(B + 1, ns),
        in_specs=[
            pl.BlockSpec(memory_space=pl.ANY),
            pl.BlockSpec((D, _E * _R + 1), lambda b, s: (0, 0)),
            pl.BlockSpec((_E, D), lambda b, s: (0, 0)),
            pl.BlockSpec((_E, 1), lambda b, s: (0, 0)),
            pl.BlockSpec((_E * _R, D), lambda b, s: (0, 0)),
            pl.BlockSpec((_E * _R, D), lambda b, s: (0, 0)),
            pl.BlockSpec((1, D), lambda b, s: (0, 0)),
            pl.BlockSpec((1, D), lambda b, s: (0, 0)),
        ],
        out_specs=pl.BlockSpec(
            (1, s_blk, D),
            lambda b, s: (jnp.maximum(b - 1, 0), s * jnp.minimum(b, 1), 0)),
        out_shape=jax.ShapeDtypeStruct((B, S, D), jnp.float32),
        scratch_shapes=[
            pltpu.VMEM((2, S, D), jnp.float32),
            pltpu.VMEM((2, 1, D), jnp.float32),
            pltpu.VMEM((2, S, _E * _R + 1), jnp.float32),
            pltpu.VMEM((2, S, 1), jnp.float32),
            pltpu.VMEM((1, D), jnp.float32),
            pltpu.VMEM((1, _E * _R), jnp.float32),
            pltpu.SMEM((1, 2), jnp.float32),
            pltpu.SemaphoreType.DMA((2, S // s_blk)),
        ],
    )(x, w65, gate_W, gb, A2, B2, gm, bt)
    return out


# speedup vs baseline: 2.2746x; 1.1340x over previous
import functools

import jax
import jax.numpy as jnp
from jax import lax
from jax.experimental import pallas as pl
from jax.experimental.pallas import tpu as pltpu

_E = 8
_R = 8
_NEG = -3.0e38


def _fused_kernel(x_ref, w65_ref, gw_ref, gb_ref, a2_ref, b2_ref,
                  gamma_ref, beta_ref, o_ref,
                  xc_ref, h_ref, xb_ref, s2_ref, delta_ref, wt_ref, st_ref,
                  sem, *, s_blk, ns, n_b, inv_s):
    b = pl.program_id(0)
    s = pl.program_id(1)
    D = x_ref.shape[2]
    slot = lax.rem(b, 2)
    pslot = 1 - slot

    def chunk_copy(bb, sl, j):
        return pltpu.make_async_copy(
            x_ref.at[bb, pl.ds(j * s_blk, s_blk), :],
            xc_ref.at[sl, pl.ds(j * s_blk, s_blk), :],
            sem.at[sl, j])

    @pl.when(jnp.logical_and(b == 0, s == 0))
    def _():
        for j in range(ns):
            chunk_copy(0, 0, j).start()

    @pl.when(jnp.logical_and(s == 0, jnp.logical_and(b >= 1, b < n_b)))
    def _():
        chunk_copy(b, slot, ns - 1).start()

    @pl.when(jnp.logical_and(s >= 1, b + 1 < n_b))
    def _():
        chunk_copy(b + 1, pslot, s - 1).start()

    @pl.when(b < n_b)
    def _stats():
        chunk_copy(b, slot, s).wait()
        xm = xc_ref[slot, pl.ds(s * s_blk, s_blk), :]

        @pl.when(s == 0)
        def _():
            h_ref[slot] = jnp.zeros((1, D), jnp.float32)

        h_ref[slot] += jnp.sum(xm, axis=0)[None, :]

        xb_ref[slot, pl.ds(s * s_blk, s_blk), :] = lax.dot_general(
            xm, w65_ref[...], (((1,), (0,)), ((), ())),
            preferred_element_type=jnp.float32)
        s2_ref[slot, pl.ds(s * s_blk, s_blk), :] = jnp.sum(
            xm * xm, axis=1, keepdims=True)

    @pl.when(b >= 1)
    def _normalize():
        @pl.when(s == 0)
        def _():
            h = h_ref[pslot] * inv_s
            logits = (jnp.sum(gw_ref[...] * h, axis=1, keepdims=True)
                      + gb_ref[...])
            iota8 = lax.broadcasted_iota(jnp.int32, (_E, 1), 0)
            m1 = jnp.max(logits)
            i1 = jnp.min(jnp.where(logits == m1, iota8, _E))
            masked = jnp.where(iota8 == i1, _NEG, logits)
            m2 = jnp.max(masked)
            i2 = jnp.min(jnp.where(masked == m2, iota8, _E))
            eb = jnp.exp(m2 - m1)
            denom = 1.0 + eb
            w1 = 1.0 / denom
            w2 = eb / denom
            t = jnp.sum(a2_ref[...] * h, axis=1, keepdims=True)
            e_ids = lax.broadcasted_iota(jnp.int32, (_E * _R, 1), 0) // _R
            wfull = (jnp.where(e_ids == i1, w1, 0.0)
                     + jnp.where(e_ids == i2, w2, 0.0))
            wt = wfull * t * (1.0 / _R)
            delta = jnp.sum(wt * b2_ref[...], axis=0, keepdims=True)
            delta_ref[...] = delta
            wt_ref[...] = wt.reshape(1, _E * _R)
            st_ref[0, 0] = jnp.sum(delta)
            st_ref[0, 1] = jnp.sum(delta * delta)

        xm = xc_ref[pslot, pl.ds(s * s_blk, s_blk), :]
        xb = xb_ref[pslot, pl.ds(s * s_blk, s_blk), :]
        s1 = xb[:, 64:65]
        cross = jnp.sum(xb[:, 0:64] * wt_ref[...], axis=1, keepdims=True)
        mu = (s1 + st_ref[0, 0]) * (1.0 / D)
        var = ((s2_ref[pslot, pl.ds(s * s_blk, s_blk), :] + 2.0 * cross
                + st_ref[0, 1]) * (1.0 / D) - mu * mu)
        rs = lax.rsqrt(var + 1e-5)
        o_ref[...] = (((xm + delta_ref[...] - mu) * rs) * gamma_ref[...]
                      + beta_ref[...])[None]


def kernel(x, gate_W, gate_b, A_all, B_all, gamma, beta):
    B, S, D = x.shape
    s_blk = 512
    ns = S // s_blk

    A2 = A_all.reshape(_E * _R, D)
    B2 = jnp.transpose(B_all, (0, 2, 1)).reshape(_E * _R, D)
    w65 = jnp.concatenate([B2.T, jnp.ones((D, 1), jnp.float32)], axis=1)
    gb = gate_b.reshape(_E, 1)
    gm = gamma.reshape(1, D)
    bt = beta.reshape(1, D)

    out = pl.pallas_call(
        functools.partial(_fused_kernel, s_blk=s_blk, ns=ns, n_b=B,
                          inv_s=1.0 / S),
        grid=(B + 1, ns),
        in_specs=[
            pl.BlockSpec(memory_space=pl.ANY),
            pl.BlockSpec((D, _E * _R + 1), lambda b, s: (0, 0)),
            pl.BlockSpec((_E, D), lambda b, s: (0, 0)),
            pl.BlockSpec((_E, 1), lambda b, s: (0, 0)),
            pl.BlockSpec((_E * _R, D), lambda b, s: (0, 0)),
            pl.BlockSpec((_E * _R, D), lambda b, s: (0, 0)),
            pl.BlockSpec((1, D), lambda b, s: (0, 0)),
            pl.BlockSpec((1, D), lambda b, s: (0, 0)),
        ],
        out_specs=pl.BlockSpec(
            (1, s_blk, D),
            lambda b, s: (jnp.maximum(b - 1, 0), s * jnp.minimum(b, 1), 0)),
        out_shape=jax.ShapeDtypeStruct((B, S, D), jnp.float32),
        scratch_shapes=[
            pltpu.VMEM((2, S, D), jnp.float32),
            pltpu.VMEM((2, 1, D), jnp.float32),
            pltpu.VMEM((2, S, _E * _R + 1), jnp.float32),
            pltpu.VMEM((2, S, 1), jnp.float32),
            pltpu.VMEM((1, D), jnp.float32),
            pltpu.VMEM((1, _E * _R), jnp.float32),
            pltpu.SMEM((1, 2), jnp.float32),
            pltpu.SemaphoreType.DMA((2, S // s_blk)),
        ],
    )(x, w65, gate_W, gb, A2, B2, gm, bt)
    return out
